# baseline (speedup 1.0000x reference)
import functools

import jax
import jax.numpy as jnp
from jax import lax
from jax.experimental import pallas as pl
from jax.experimental.pallas import tpu as pltpu
from jax.experimental.pallas import tpu_sc as plsc

SEQ = 200
BATCH = 4096
EMBED = 64
OUT = 32
HALF = SEQ // 2


def _make_sc_pool(num_cores, num_subcores):
    nw = num_cores * num_subcores
    b_per_w = BATCH // nw
    mesh = plsc.VectorSubcoreMesh(
        core_axis_name="c", subcore_axis_name="s",
        num_cores=num_cores, num_subcores=num_subcores)

    @functools.partial(
        pl.kernel,
        mesh=mesh,
        out_type=jax.ShapeDtypeStruct((BATCH, EMBED), jnp.float32),
        scratch_types=[
            pltpu.VMEM((b_per_w, 2, HALF), jnp.int32),
            pltpu.VMEM((SEQ, EMBED), jnp.float32),
            pltpu.VMEM((SEQ, EMBED), jnp.float32),
            pltpu.VMEM((b_per_w, EMBED), jnp.float32),
            pltpu.SemaphoreType.DMA,
            pltpu.SemaphoreType.DMA,
        ],
        compiler_params=pltpu.CompilerParams(use_tc_tiling_on_sc=False),
    )
    def sc_pool(x3_hbm, table_hbm, out_hbm, idx_v, rows_a, rows_b, acc_v,
                sem_a, sem_b):
        wid = lax.axis_index("s") * num_cores + lax.axis_index("c")
        base = wid * b_per_w

        pltpu.sync_copy(x3_hbm.at[pl.ds(base, b_per_w)], idx_v)

        def gather(b, rows, sem):
            pltpu.async_copy(table_hbm.at[idx_v.at[b, 0]],
                             rows.at[pl.ds(0, HALF)], sem)
            pltpu.async_copy(table_hbm.at[idx_v.at[b, 1]],
                             rows.at[pl.ds(HALF, HALF)], sem)

        def wait(rows, sem):
            pltpu.make_async_copy(table_hbm.at[pl.ds(0, SEQ)], rows, sem).wait()

        def accum(rows, b):
            u = 8
            z = jnp.zeros((16,), jnp.float32)

            def body(i, carry):
                a0, a1, a2, a3 = carry
                s = i * u
                for k in range(u):
                    a0 = a0 + rows[s + k, pl.ds(0, 16)]
                    a1 = a1 + rows[s + k, pl.ds(16, 16)]
                    a2 = a2 + rows[s + k, pl.ds(32, 16)]
                    a3 = a3 + rows[s + k, pl.ds(48, 16)]
                return (a0, a1, a2, a3)

            a0, a1, a2, a3 = lax.fori_loop(0, SEQ // u, body, (z, z, z, z))
            acc_v[b, pl.ds(0, 16)] = a0
            acc_v[b, pl.ds(16, 16)] = a1
            acc_v[b, pl.ds(32, 16)] = a2
            acc_v[b, pl.ds(48, 16)] = a3

        gather(0, rows_a, sem_a)
        gather(1, rows_b, sem_b)

        def pair(i, _):
            b0 = 2 * i
            wait(rows_a, sem_a)

            @pl.when(b0 + 2 < b_per_w)
            def _():
                gather(b0 + 2, rows_a, sem_a)

            accum(rows_a, b0)
            wait(rows_b, sem_b)

            @pl.when(b0 + 3 < b_per_w)
            def _():
                gather(b0 + 3, rows_b, sem_b)

            accum(rows_b, b0 + 1)
            return 0

        lax.fori_loop(0, b_per_w // 2, pair, 0)

        pltpu.sync_copy(acc_v, out_hbm.at[pl.ds(base, b_per_w)])

    return sc_pool


def _linear_body(p_ref, w_ref, b_ref, o_ref):
    p = p_ref[...]
    w = w_ref[...]
    acc = lax.dot_general(p, w, (((1,), (1,)), ((), ())),
                          preferred_element_type=jnp.float32)
    o_ref[...] = acc * (1.0 / SEQ) + b_ref[...]


def _linear(pooled_sum, fc_w, fc_b2):
    blk = 512
    return pl.pallas_call(
        _linear_body,
        grid=(BATCH // blk,),
        in_specs=[
            pl.BlockSpec((blk, EMBED), lambda i: (i, 0)),
            pl.BlockSpec((OUT, EMBED), lambda i: (0, 0)),
            pl.BlockSpec((1, OUT), lambda i: (0, 0)),
        ],
        out_specs=pl.BlockSpec((blk, OUT), lambda i: (i, 0)),
        out_shape=jax.ShapeDtypeStruct((BATCH, OUT), jnp.float32),
    )(pooled_sum, fc_w, fc_b2)


def kernel(x, emb_table, fc_w, fc_b):
    x3 = jnp.transpose(x.astype(jnp.int32)).reshape(BATCH, 2, HALF)
    info = plsc.get_sparse_core_info()
    sc_pool = _make_sc_pool(info.num_cores, info.num_subcores)
    pooled_sum = sc_pool(x3, emb_table)
    return _linear(pooled_sum, fc_w, fc_b.reshape(1, OUT))
